# Initial kernel scaffold; baseline (speedup 1.0000x reference)
import jax
import jax.numpy as jnp
from jax.experimental import pallas as pl


def kernel(z, edge_index):
    raise NotImplementedError("write your pallas kernel here")


# speedup vs baseline: 3.5823x; 3.5823x over previous
import functools

import jax
import jax.numpy as jnp
from jax import lax
from jax.experimental import pallas as pl
from jax.experimental.pallas import tpu as pltpu
from jax.experimental.pallas import tpu_sc as plsc

E = 320000
D = 128
L = 16
NC = 2
NS = 16
NW = NC * NS
EPW = E // NW
C = 80
NCHUNK = EPW // C

_mesh = plsc.VectorSubcoreMesh(core_axis_name="c", subcore_axis_name="s")


@functools.partial(
    pl.kernel,
    out_type=jax.ShapeDtypeStruct((E,), jnp.float32),
    mesh=_mesh,
    scratch_types=[
        pltpu.VMEM((C,), jnp.int32),
        pltpu.VMEM((C,), jnp.int32),
        pltpu.VMEM((C, D), jnp.float32),
        pltpu.VMEM((C, D), jnp.float32),
        pltpu.VMEM((EPW,), jnp.float32),
        pltpu.VMEM((L * L,), jnp.float32),
        pltpu.SemaphoreType.DMA,
        pltpu.SemaphoreType.DMA,
    ],
    compiler_params=pltpu.CompilerParams(needs_layout_passes=False),
)
def _decode(z_hbm, src_hbm, dst_hbm, out_hbm,
            sidx, didx, srows, drows, oval, tstage, sem_s, sem_d):
    wid = lax.axis_index("s") * NC + lax.axis_index("c")
    base = wid * EPW

    def chunk_body(ci, carry):
        off = base + ci * C
        pltpu.sync_copy(src_hbm.at[pl.ds(off, C)], sidx)
        pltpu.sync_copy(dst_hbm.at[pl.ds(off, C)], didx)
        cp_s = pltpu.async_copy(z_hbm.at[sidx], srows, sem_s)
        cp_d = pltpu.async_copy(z_hbm.at[didx], drows, sem_d)
        cp_s.wait()
        cp_d.wait()

        def group_body(g, carry2):
            lanes = lax.iota(jnp.int32, L)
            for e in range(L):
                row = g * L + e
                acc = srows[row, pl.ds(0, L)] * drows[row, pl.ds(0, L)]
                for k in range(1, D // L):
                    acc = acc + (srows[row, pl.ds(k * L, L)]
                                 * drows[row, pl.ds(k * L, L)])
                plsc.store_scatter(tstage, [lanes * L + e], acc)
            dots = tstage[pl.ds(0, L)]
            for r in range(1, L):
                dots = dots + tstage[pl.ds(r * L, L)]
            oval[pl.ds(ci * C + g * L, L)] = 1.0 / (1.0 + jnp.exp(-dots))
            return carry2

        return lax.fori_loop(0, C // L, group_body, carry)

    lax.fori_loop(0, NCHUNK, chunk_body, 0)
    pltpu.sync_copy(oval, out_hbm.at[pl.ds(base, EPW)])


def kernel(z, edge_index):
    ei = edge_index.astype(jnp.int32)
    return _decode(z, ei[0], ei[1])


# speedup vs baseline: 7.8826x; 2.2004x over previous
import functools

import jax
import jax.numpy as jnp
from jax import lax
from jax.experimental import pallas as pl
from jax.experimental.pallas import tpu as pltpu
from jax.experimental.pallas import tpu_sc as plsc

E = 320000
D = 128
L = 16
NC = 2
NS = 16
NW = NC * NS
EPW = E // NW
C = 80
NCHUNK = EPW // C
NPAIR = (NCHUNK - 1) // 2

_mesh = plsc.VectorSubcoreMesh(core_axis_name="c", subcore_axis_name="s")


@functools.partial(
    pl.kernel,
    out_type=jax.ShapeDtypeStruct((E,), jnp.float32),
    mesh=_mesh,
    scratch_types=[
        pltpu.VMEM((EPW,), jnp.int32),
        pltpu.VMEM((EPW,), jnp.int32),
        pltpu.VMEM((C, D), jnp.float32),
        pltpu.VMEM((C, D), jnp.float32),
        pltpu.VMEM((C, D), jnp.float32),
        pltpu.VMEM((C, D), jnp.float32),
        pltpu.VMEM((EPW,), jnp.float32),
        pltpu.VMEM((L * L,), jnp.float32),
        pltpu.SemaphoreType.DMA,
        pltpu.SemaphoreType.DMA,
        pltpu.SemaphoreType.DMA,
        pltpu.SemaphoreType.DMA,
    ],
    compiler_params=pltpu.CompilerParams(needs_layout_passes=False),
)
def _decode(z_hbm, src_hbm, dst_hbm, out_hbm,
            sidx, didx, srows_a, drows_a, srows_b, drows_b, oval, tstage,
            sem_sa, sem_da, sem_sb, sem_db):
    wid = lax.axis_index("s") * NC + lax.axis_index("c")
    base = wid * EPW
    pltpu.sync_copy(src_hbm.at[pl.ds(base, EPW)], sidx)
    pltpu.sync_copy(dst_hbm.at[pl.ds(base, EPW)], didx)

    def start(ci, srows, drows, sem_s, sem_d):
        pltpu.async_copy(z_hbm.at[sidx.at[pl.ds(ci * C, C)]], srows, sem_s)
        pltpu.async_copy(z_hbm.at[didx.at[pl.ds(ci * C, C)]], drows, sem_d)

    def wait(srows, drows, sem_s, sem_d):
        pltpu.make_async_copy(z_hbm.at[sidx.at[pl.ds(0, C)]], srows, sem_s).wait()
        pltpu.make_async_copy(z_hbm.at[didx.at[pl.ds(0, C)]], drows, sem_d).wait()

    def compute(ci, srows, drows):
        def group_body(g, carry2):
            lanes = lax.iota(jnp.int32, L)
            for e in range(L):
                row = g * L + e
                acc = srows[row, pl.ds(0, L)] * drows[row, pl.ds(0, L)]
                for k in range(1, D // L):
                    acc = acc + (srows[row, pl.ds(k * L, L)]
                                 * drows[row, pl.ds(k * L, L)])
                plsc.store_scatter(tstage, [lanes * L + e], acc)
            dots = tstage[pl.ds(0, L)]
            for r in range(1, L):
                dots = dots + tstage[pl.ds(r * L, L)]
            oval[pl.ds(ci * C + g * L, L)] = 1.0 / (1.0 + jnp.exp(-dots))
            return carry2

        lax.fori_loop(0, C // L, group_body, 0)

    start(0, srows_a, drows_a, sem_sa, sem_da)

    def pair_body(p, carry):
        c0 = 2 * p
        start(c0 + 1, srows_b, drows_b, sem_sb, sem_db)
        wait(srows_a, drows_a, sem_sa, sem_da)
        compute(c0, srows_a, drows_a)
        start(c0 + 2, srows_a, drows_a, sem_sa, sem_da)
        wait(srows_b, drows_b, sem_sb, sem_db)
        compute(c0 + 1, srows_b, drows_b)
        return carry

    lax.fori_loop(0, NPAIR, pair_body, 0)
    wait(srows_a, drows_a, sem_sa, sem_da)
    compute(NCHUNK - 1, srows_a, drows_a)

    pltpu.sync_copy(oval, out_hbm.at[pl.ds(base, EPW)])


def kernel(z, edge_index):
    ei = edge_index.astype(jnp.int32)
    return _decode(z, ei[0], ei[1])
